# baseline (speedup 1.0000x reference)
import functools

import jax
import jax.numpy as jnp
from jax import lax
from jax.experimental import pallas as pl
from jax.experimental.pallas import tpu as pltpu
from jax.experimental.pallas import tpu_sc as plsc

EMBED_DIM = 128
SEQLEN = 2048
BATCH = 4
TOTAL = BATCH * SEQLEN
NUM_CORES = 2
NUM_SUBCORES = 16
NUM_WORKERS = NUM_CORES * NUM_SUBCORES
ROWS_PER_W = TOTAL // NUM_WORKERS
CHUNK = 128
NCHUNK = ROWS_PER_W // CHUNK


@functools.partial(
    pl.kernel,
    mesh=plsc.VectorSubcoreMesh(core_axis_name="c", subcore_axis_name="s"),
    out_type=jax.ShapeDtypeStruct((TOTAL, EMBED_DIM), jnp.float32),
    scratch_types=[
        pltpu.VMEM((1, ROWS_PER_W), jnp.int32),
        pltpu.VMEM((ROWS_PER_W, EMBED_DIM), jnp.float32),
        pltpu.VMEM_SHARED((SEQLEN, EMBED_DIM), jnp.float32),
        pltpu.SemaphoreType.DMA,
    ],
)
def _embed_kernel(ids_hbm, word_hbm, pos_hbm, out_hbm, idx_v, rows_v, pos_sh, sem):
    wid = lax.axis_index("s") * NUM_CORES + lax.axis_index("c")
    sid = lax.axis_index("s")
    base = wid * ROWS_PER_W
    pbase = lax.rem(base, SEQLEN)

    pltpu.sync_copy(
        pos_hbm.at[pl.ds(sid * (SEQLEN // NUM_SUBCORES), SEQLEN // NUM_SUBCORES)],
        pos_sh.at[pl.ds(sid * (SEQLEN // NUM_SUBCORES), SEQLEN // NUM_SUBCORES)],
    )
    pltpu.sync_copy(ids_hbm.at[wid], idx_v)
    plsc.subcore_barrier()
    pltpu.sync_copy(pos_sh.at[pl.ds(pbase, ROWS_PER_W)], rows_v)

    pltpu.async_copy(word_hbm.at[idx_v.at[0]], rows_v, sem, add=True).wait()

    pltpu.sync_copy(rows_v, out_hbm.at[pl.ds(base, ROWS_PER_W)])


def kernel(input_ids, word_table, pos_table):
    ids = input_ids.reshape(NUM_WORKERS, 1, ROWS_PER_W).astype(jnp.int32)
    out = _embed_kernel(ids, word_table, pos_table)
    return out.reshape(BATCH, SEQLEN, EMBED_DIM)


# speedup vs baseline: 1.0537x; 1.0537x over previous
import functools

import jax
import jax.numpy as jnp
from jax import lax
from jax.experimental import pallas as pl
from jax.experimental.pallas import tpu as pltpu
from jax.experimental.pallas import tpu_sc as plsc

EMBED_DIM = 128
SEQLEN = 2048
BATCH = 4
TOTAL = BATCH * SEQLEN
NUM_CORES = 2
NUM_SUBCORES = 16
NUM_WORKERS = NUM_CORES * NUM_SUBCORES
ROWS_PER_W = TOTAL // NUM_WORKERS
CHUNK = 128
NCHUNK = ROWS_PER_W // CHUNK


@functools.partial(
    pl.kernel,
    mesh=plsc.VectorSubcoreMesh(core_axis_name="c", subcore_axis_name="s"),
    out_type=jax.ShapeDtypeStruct((TOTAL, EMBED_DIM), jnp.float32),
    scratch_types=[
        pltpu.VMEM((1, ROWS_PER_W), jnp.int32),
        pltpu.VMEM((ROWS_PER_W, EMBED_DIM), jnp.float32),
        pltpu.SemaphoreType.DMA,
        pltpu.SemaphoreType.DMA,
    ],
)
def _embed_kernel(ids_hbm, word_hbm, pos_hbm, out_hbm, idx_v, rows_v, sem, sem2):
    wid = lax.axis_index("s") * NUM_CORES + lax.axis_index("c")
    base = wid * ROWS_PER_W
    pbase = lax.rem(base, SEQLEN)

    idx_cp = pltpu.async_copy(ids_hbm.at[wid], idx_v, sem2)
    pltpu.sync_copy(pos_hbm.at[pl.ds(pbase, ROWS_PER_W)], rows_v)
    idx_cp.wait()

    pltpu.async_copy(word_hbm.at[idx_v.at[0]], rows_v, sem, add=True).wait()

    pltpu.sync_copy(rows_v, out_hbm.at[pl.ds(base, ROWS_PER_W)])


def kernel(input_ids, word_table, pos_table):
    ids = input_ids.reshape(NUM_WORKERS, 1, ROWS_PER_W).astype(jnp.int32)
    out = _embed_kernel(ids, word_table, pos_table)
    return out.reshape(BATCH, SEQLEN, EMBED_DIM)


# speedup vs baseline: 1.0552x; 1.0013x over previous
import functools

import jax
import jax.numpy as jnp
from jax import lax
from jax.experimental import pallas as pl
from jax.experimental.pallas import tpu as pltpu
from jax.experimental.pallas import tpu_sc as plsc

EMBED_DIM = 128
SEQLEN = 2048
BATCH = 4
TOTAL = BATCH * SEQLEN
NUM_CORES = 2
NUM_SUBCORES = 16
NUM_WORKERS = NUM_CORES * NUM_SUBCORES
ROWS_PER_W = TOTAL // NUM_WORKERS
CHUNK = 128
NCHUNK = ROWS_PER_W // CHUNK


@functools.partial(
    pl.kernel,
    mesh=plsc.VectorSubcoreMesh(core_axis_name="c", subcore_axis_name="s"),
    out_type=jax.ShapeDtypeStruct((TOTAL, EMBED_DIM), jnp.float32),
    scratch_types=[
        pltpu.VMEM((1, ROWS_PER_W), jnp.int32),
        pltpu.VMEM((ROWS_PER_W, EMBED_DIM), jnp.float32),
        pltpu.SemaphoreType.DMA,
    ],
)
def _embed_kernel(ids_hbm, word_hbm, pos_hbm, out_hbm, idx_v, rows_v, sem):
    wid = lax.axis_index("s") * NUM_CORES + lax.axis_index("c")
    base = wid * ROWS_PER_W
    pbase = lax.rem(base, SEQLEN)

    idx_cp = pltpu.async_copy(ids_hbm.at[wid], idx_v, sem)
    pre_cp = pltpu.async_copy(pos_hbm.at[pl.ds(pbase, ROWS_PER_W)], rows_v, sem)
    idx_cp.wait()
    pre_cp.wait()

    pltpu.async_copy(word_hbm.at[idx_v.at[0]], rows_v, sem, add=True).wait()

    pltpu.sync_copy(rows_v, out_hbm.at[pl.ds(base, ROWS_PER_W)])


def kernel(input_ids, word_table, pos_table):
    ids = input_ids.reshape(NUM_WORKERS, 1, ROWS_PER_W).astype(jnp.int32)
    out = _embed_kernel(ids, word_table, pos_table)
    return out.reshape(BATCH, SEQLEN, EMBED_DIM)


# speedup vs baseline: 1.0571x; 1.0018x over previous
import functools

import jax
import jax.numpy as jnp
from jax import lax
from jax.experimental import pallas as pl
from jax.experimental.pallas import tpu as pltpu
from jax.experimental.pallas import tpu_sc as plsc

EMBED_DIM = 128
SEQLEN = 2048
BATCH = 4
TOTAL = BATCH * SEQLEN
NUM_CORES = 2
NUM_SUBCORES = 16
NUM_WORKERS = NUM_CORES * NUM_SUBCORES
ROWS_PER_W = TOTAL // NUM_WORKERS


@functools.partial(
    pl.kernel,
    mesh=plsc.VectorSubcoreMesh(core_axis_name="c", subcore_axis_name="s"),
    out_type=jax.ShapeDtypeStruct((TOTAL, EMBED_DIM), jnp.float32),
    scratch_types=[
        pltpu.VMEM((1, ROWS_PER_W), jnp.int32),
        pltpu.VMEM((ROWS_PER_W, EMBED_DIM), jnp.float32),
        pltpu.SemaphoreType.DMA,
    ],
)
def _embed_kernel(ids_hbm, word_hbm, pos_hbm, out_hbm, idx_v, rows_v, sem):
    wid = lax.axis_index("s") * NUM_CORES + lax.axis_index("c")
    base = wid * ROWS_PER_W
    b = wid // (SEQLEN // ROWS_PER_W)
    pbase = lax.rem(base, SEQLEN)

    idx_cp = pltpu.async_copy(ids_hbm.at[b, pl.ds(pbase, ROWS_PER_W)], idx_v.at[0], sem)
    pre_cp = pltpu.async_copy(pos_hbm.at[pl.ds(pbase, ROWS_PER_W)], rows_v, sem)
    idx_cp.wait()
    pre_cp.wait()

    pltpu.async_copy(word_hbm.at[idx_v.at[0]], rows_v, sem, add=True).wait()

    pltpu.sync_copy(rows_v, out_hbm.at[pl.ds(base, ROWS_PER_W)])


def kernel(input_ids, word_table, pos_table):
    out = _embed_kernel(input_ids.astype(jnp.int32), word_table, pos_table)
    return out.reshape(BATCH, SEQLEN, EMBED_DIM)
